# Initial kernel scaffold; baseline (speedup 1.0000x reference)
import jax
import jax.numpy as jnp
from jax.experimental import pallas as pl


def kernel(x, W_up, W_down, expert_bias):
    raise NotImplementedError("write your pallas kernel here")


# speedup vs baseline: 4.5988x; 4.5988x over previous
import functools

import jax
import jax.numpy as jnp
from jax.experimental import pallas as pl

_T, _B, _S, _D = 4, 1, 2048, 1024
_E = 8
_TOPK = 2
_NC = 64
_DFF = 4096
_EF = _DFF // _E
_CPE = _NC // _E
_BETA = 0.5
_THRESH = 1.0


def _routing_kernel(x_ref, bias_ref, w_ref, cnt_ref, rps_ref):
    i = pl.program_id(0)
    tn = x_ref.shape[1]
    d = x_ref.shape[2]
    v = jnp.zeros((tn, d), jnp.float32)
    fr = jnp.zeros((tn, d), jnp.float32)
    for t in range(_T):
        v = _BETA * v + x_ref[t]
        s = (v >= _THRESH).astype(jnp.float32)
        fr = fr + s
        v = v - s * _THRESH
    fr = fr * (1.0 / _T)
    es = fr.reshape(tn, d // _NC, _E, _CPE).sum(axis=3).sum(axis=1)
    es = es * (1.0 / (_D // _NC) / _CPE)
    es = es + bias_ref[0][None, :]
    eidx = jax.lax.broadcasted_iota(jnp.int32, (tn, _E), 1)
    m1 = jnp.max(es, axis=1)
    i1 = jnp.min(jnp.where(es == m1[:, None], eidx, _E), axis=1)
    masked = jnp.where(eidx == i1[:, None], -jnp.inf, es)
    m2 = jnp.max(masked, axis=1)
    i2 = jnp.min(jnp.where(masked == m2[:, None], eidx, _E), axis=1)
    eb = jnp.exp(m2 - m1)
    w1 = 1.0 / (1.0 + eb)
    w2 = eb / (1.0 + eb)
    eidx_t = jax.lax.broadcasted_iota(jnp.int32, (_E, tn), 0)
    w_et = jnp.where(eidx_t == i1[None, :], w1[None, :], 0.0)
    w_et = w_et + jnp.where(eidx_t == i2[None, :], w2[None, :], 0.0)
    w_ref[...] = w_et[:, None, :]
    cnt = (jnp.sum((eidx == i1[:, None]).astype(jnp.float32), axis=0)
           + jnp.sum((eidx == i2[:, None]).astype(jnp.float32), axis=0))
    ex = jnp.exp(es - m1[:, None])
    rp = ex / jnp.sum(ex, axis=1, keepdims=True)
    rps = jnp.sum(rp, axis=0)

    @pl.when(i == 0)
    def _():
        cnt_ref[...] = cnt[None, :]
        rps_ref[...] = rps[None, :]

    @pl.when(i > 0)
    def _():
        cnt_ref[...] += cnt[None, :]
        rps_ref[...] += rps[None, :]


def _expert_kernel(x_ref, w_ref, wup_ref, wdn_ref, out_ref):
    e = pl.program_id(1)
    tm = x_ref.shape[1]
    wup = wup_ref[0]
    wdn = wdn_ref[0]
    v = jnp.zeros((tm, _EF), jnp.float32)
    h = []
    for t in range(_T):
        u = jax.lax.dot_general(x_ref[t], wup, (((1,), (1,)), ((), ())),
                                preferred_element_type=jnp.float32)
        v = _BETA * v + u
        s = (v >= _THRESH).astype(jnp.float32)
        h.append(s)
        v = v - s * _THRESH
    v2 = jnp.zeros((tm, _D), jnp.float32)
    wcol = w_ref[0, 0][:, None]
    for t in range(_T):
        o = jax.lax.dot_general(h[t], wdn, (((1,), (1,)), ((), ())),
                                preferred_element_type=jnp.float32)
        v2 = _BETA * v2 + o
        s2 = (v2 >= _THRESH).astype(jnp.float32)
        v2 = v2 - s2 * _THRESH
        contrib = s2 * wcol

        @pl.when(e == 0)
        def _():
            out_ref[t] = contrib

        @pl.when(e > 0)
        def _():
            out_ref[t] += contrib


def kernel(x, W_up, W_down, expert_bias):
    Tt, Bb, Ss, Dd = x.shape
    N = Bb * Ss
    xf = x.reshape(Tt, N, Dd)
    bias2d = expert_bias.reshape(1, _E)

    TN = 512
    w_et, cnt, rps = pl.pallas_call(
        _routing_kernel,
        grid=(N // TN,),
        in_specs=[
            pl.BlockSpec((Tt, TN, Dd), lambda i: (0, i, 0)),
            pl.BlockSpec((1, _E), lambda i: (0, 0)),
        ],
        out_specs=[
            pl.BlockSpec((_E, 1, TN), lambda i: (0, 0, i)),
            pl.BlockSpec((1, _E), lambda i: (0, 0)),
            pl.BlockSpec((1, _E), lambda i: (0, 0)),
        ],
        out_shape=[
            jax.ShapeDtypeStruct((_E, 1, N), jnp.float32),
            jax.ShapeDtypeStruct((1, _E), jnp.float32),
            jax.ShapeDtypeStruct((1, _E), jnp.float32),
        ],
    )(xf, bias2d)

    TM = 256
    out = pl.pallas_call(
        _expert_kernel,
        grid=(N // TM, _E),
        in_specs=[
            pl.BlockSpec((Tt, TM, Dd), lambda i, e: (0, i, 0)),
            pl.BlockSpec((1, 1, TM), lambda i, e: (e, 0, i)),
            pl.BlockSpec((1, _EF, Dd), lambda i, e: (e, 0, 0)),
            pl.BlockSpec((1, Dd, _EF), lambda i, e: (e, 0, 0)),
        ],
        out_specs=pl.BlockSpec((Tt, TM, Dd), lambda i, e: (0, i, 0)),
        out_shape=jax.ShapeDtypeStruct((Tt, N, Dd), jnp.float32),
    )(xf, w_et, W_up, W_down)

    ef_frac = cnt[0] / (N * _TOPK)
    rp = rps[0] / N
    lb = _E * jnp.sum(ef_frac * rp)
    return out.reshape(Tt, Bb, Ss, Dd), lb
